# baseline (speedup 1.0000x reference)
import functools

import jax
import jax.numpy as jnp
from jax.experimental import pallas as pl
from jax.experimental.pallas import tpu as pltpu

V = 27
VP = 32
NCLS = 60
LCFG = [(3, 64, 1, False), (64, 64, 1, True), (64, 64, 1, True), (64, 64, 1, True),
        (64, 128, 2, True), (128, 128, 1, True), (128, 128, 1, True),
        (128, 256, 2, True), (256, 256, 1, True), (256, 256, 1, True)]
G = 4


def _pad_rows(w, rp):
    return jnp.pad(w, ((0, rp - w.shape[0]), (0, 0)))


def _prep(params):
    consts = []

    def add(a):
        consts.append(a.astype(jnp.float32))
        return len(consts) - 1

    def addh(a):
        consts.append(a.astype(jnp.bfloat16))
        return len(consts) - 1

    g = jnp.pad(params['data_bn']['g'].reshape(2, V, 3), ((0, 0), (0, VP - V), (0, 5)))
    b = jnp.pad(params['data_bn']['b'].reshape(2, V, 3), ((0, 0), (0, VP - V), (0, 5)))
    ig = add(jnp.tile(g, (G // 2, 1, 1)).reshape(G, 1, VP, 8))
    ib = add(jnp.tile(b, (G // 2, 1, 1)).reshape(G, 1, VP, 8))

    meta = []
    for (cin, cout, stride, residual), lp in zip(LCFG, params['layers']):
        gc, tc = lp['gcn'], lp['tcn']
        cp = 8 if cin == 3 else cin
        E = cout // 4
        m = {}
        wab = jnp.concatenate([_pad_rows(w, cp) for w in (gc['wa'] + gc['wb'])], axis=1)
        bab = jnp.concatenate(gc['ba'] + gc['bb']).reshape(1, 6 * E)
        m['wab'] = addh(wab)
        m['bab'] = add(bab)
        m['A'] = add(jnp.pad(gc['PA'], ((0, 0), (0, VP - V), (0, VP - V))))
        bng, bnb = gc['bn_g'], gc['bn_b']
        wd = jnp.concatenate([_pad_rows(w, cp) for w in gc['wd']], axis=0) * bng[None, :]
        bd = ((gc['bd'][0] + gc['bd'][1] + gc['bd'][2]) * bng + bnb).reshape(1, cout)
        m['wd'] = addh(wd)
        m['bd'] = add(bd)
        has_down = 'down_w' in gc
        if has_down:
            m['dw'] = addh(_pad_rows(gc['down_w'], cp) * gc['down_g'][None, :])
            m['db'] = add((gc['down_b'] * gc['down_g'] + gc['down_bb']).reshape(1, cout))
        tg = tc['g']
        wt = jnp.concatenate([tc['w'][:, :, k, 0].T * tg[None, :] for k in range(9)], axis=0)
        m['wt'] = addh(wt)
        m['bt'] = add((tc['b'] * tg + tc['bb']).reshape(1, cout))
        has_res = 'res' in lp
        if has_res:
            r = lp['res']
            m['rw'] = addh(r['w'][:, :, 0, 0].T * r['g'][None, :])
            m['rb'] = add((r['b'] * r['g'] + r['bb']).reshape(1, cout))
        meta.append((cp, cout, stride, residual, has_down, has_res, m))
    ifw = addh(params['fc_w'])
    ifb = add(params['fc_b'].reshape(1, NCLS))
    return consts, tuple(meta), (ig, ib, ifw, ifb)


def _body(meta, misc, x_ref, *refs):
    out_ref = refs[-1]
    cr = refs[:-1]
    ig, ib, ifw, ifb = misc
    B = G
    T = 64

    rowmask = (jax.lax.broadcasted_iota(jnp.int32, (VP, 1), 0) >= V)
    neg = jnp.where(rowmask, jnp.float32(-1e30), jnp.float32(0.0))
    colmask = (jax.lax.broadcasted_iota(jnp.int32, (1, VP), 1) < V).astype(jnp.float32)

    act = x_ref[...] * cr[ig][...] + cr[ib][...]

    for (cp, cout, stride, residual, has_down, has_res, m) in meta:
        E = cout // 4
        rows = B * T * VP
        X2 = act.reshape(rows, cp)
        X2h = X2.astype(jnp.bfloat16)
        a12 = jnp.dot(X2h, cr[m['wab']][...], preferred_element_type=jnp.float32)
        a12 = (a12 + cr[m['bab']][...]).astype(jnp.bfloat16)
        a12 = jnp.swapaxes(a12.reshape(B, T, VP, 6 * E), 2, 3).reshape(B, T, 6, E, VP)
        Apad = cr[m['A']][...]
        act_vm = jnp.swapaxes(X2h.reshape(B, T, VP, cp), 2, 3)
        Xv = act_vm.reshape(B, T * cp, VP)
        a1cat = jnp.concatenate([a12[:, :, i] for i in range(3)],
                                axis=-1).reshape(B, T * E, 3 * VP)
        a2cat = jnp.concatenate([a12[:, :, 3 + i] for i in range(3)],
                                axis=-1).reshape(B, T * E, 3 * VP)
        attbig = jax.lax.dot_general(a1cat, a2cat, (((1,), (1,)), ((0,), (0,))),
                                     preferred_element_type=jnp.float32)
        attbig = attbig * jnp.float32(1.0 / (E * T))
        A1s = []
        for i in range(3):
            att = attbig[:, VP * i:VP * (i + 1), VP * i:VP * (i + 1)] + neg
            mx = jnp.max(att, axis=1, keepdims=True)
            e = jnp.exp(att - mx)
            sm = e / jnp.sum(e, axis=1, keepdims=True)
            A1s.append(sm * colmask + Apad[i][None])
        A1cat = jnp.concatenate(A1s, axis=-1).astype(jnp.bfloat16)
        xab = jax.lax.dot_general(Xv, A1cat, (((2,), (1,)), ((0,), (0,))),
                                  preferred_element_type=jnp.float32)
        xab = xab.astype(jnp.bfloat16).reshape(B, T, cp, 3 * VP)
        xat = jnp.swapaxes(xab, 2, 3).reshape(B, T, 3, VP, cp)
        wd_all = cr[m['wd']][...]
        y = cr[m['bd']][...] * jnp.ones((rows, 1), jnp.float32)
        for i in range(3):
            xi = xat[:, :, i].reshape(rows, cp)
            y = y + jnp.dot(xi, wd_all[i * cp:(i + 1) * cp],
                            preferred_element_type=jnp.float32)
        if has_down:
            res = jnp.dot(X2h, cr[m['dw']][...], preferred_element_type=jnp.float32)
            res = res + cr[m['db']][...]
        else:
            res = X2
        gout = jax.nn.relu(y + res).astype(jnp.bfloat16).reshape(B, T, VP, cout)

        To = T // stride
        zer = jnp.zeros((B, 4, VP, cout), jnp.bfloat16)
        xp = jnp.concatenate([zer, gout, zer], axis=1)
        wt = cr[m['wt']][...]
        acc = cr[m['bt']][...].astype(jnp.float32) * jnp.ones((B * To * VP, 1), jnp.float32)
        if stride == 1:
            tap = [xp[:, k:k + T] for k in range(9)]
        else:
            xp2 = xp.reshape(B, (T + 8) // 2, 2, VP, cout)
            tap = [xp2[:, k // 2:k // 2 + To, k % 2] for k in range(9)]
        grp = 4 if cout == 64 else 2
        for kp in range(8 // grp):
            sl = jnp.concatenate(tap[grp * kp:grp * (kp + 1)],
                                 axis=-1).reshape(B * To * VP, grp * cout)
            acc = acc + jnp.dot(sl, wt[grp * kp * cout:grp * (kp + 1) * cout],
                                preferred_element_type=jnp.float32)
        acc = acc + jnp.dot(tap[8].reshape(B * To * VP, cout), wt[8 * cout:9 * cout],
                            preferred_element_type=jnp.float32)

        if not residual:
            nxt = jax.nn.relu(acc)
        elif has_res:
            if stride == 2:
                xs = act.reshape(B, T // 2, 2, VP, cp)[:, :, 0]
            else:
                xs = act
            res2 = jnp.dot(xs.reshape(B * To * VP, cp).astype(jnp.bfloat16),
                           cr[m['rw']][...],
                           preferred_element_type=jnp.float32) + cr[m['rb']][...]
            nxt = jax.nn.relu(acc + res2)
        else:
            nxt = jax.nn.relu(acc + X2)
        act = nxt.reshape(B, To, VP, cout)
        T = To

    vmask = (jax.lax.broadcasted_iota(jnp.int32, (VP, 1), 0) < V).astype(jnp.float32)
    s = jnp.sum(act * vmask, axis=1)
    s = jnp.sum(s, axis=1) * jnp.float32(1.0 / (T * V))
    pr = s.reshape(B // 2, 2, 256)
    pm = (pr[:, 0] + pr[:, 1]) * jnp.float32(0.5)
    out = jnp.dot(pm.astype(jnp.bfloat16), cr[ifw][...], preferred_element_type=jnp.float32)
    out_ref[0] = out + cr[ifb][...]


def kernel(x, params):
    consts, meta, misc = _prep(params)
    N, C0, T0, V0, M = x.shape
    xp = jnp.transpose(x, (0, 4, 2, 3, 1)).reshape(N * M, T0, V0, C0)
    xp = jnp.pad(xp, ((0, 0), (0, 0), (0, VP - V0), (0, 8 - C0)))
    B = N * M
    in_specs = [pl.BlockSpec((G, T0, VP, 8), lambda i: (i, 0, 0, 0))]
    for c in consts:
        nd = c.ndim
        in_specs.append(pl.BlockSpec(c.shape, (lambda nd_: lambda i: (0,) * nd_)(nd)))
    out = pl.pallas_call(
        functools.partial(_body, meta, misc),
        grid=(B // G,),
        in_specs=in_specs,
        out_specs=pl.BlockSpec((1, G // 2, NCLS), lambda i: (i, 0, 0)),
        out_shape=jax.ShapeDtypeStruct((B // G, G // 2, NCLS), jnp.float32),
        compiler_params=pltpu.CompilerParams(dimension_semantics=("arbitrary",)),
    )(xp, *consts)
    return out.reshape(N, NCLS)


# speedup vs baseline: 1.1197x; 1.1197x over previous
import functools

import jax
import jax.numpy as jnp
from jax.experimental import pallas as pl
from jax.experimental.pallas import tpu as pltpu

V = 27
VP = 32
NCLS = 60
LCFG = [(3, 64, 1, False), (64, 64, 1, True), (64, 64, 1, True), (64, 64, 1, True),
        (64, 128, 2, True), (128, 128, 1, True), (128, 128, 1, True),
        (128, 256, 2, True), (256, 256, 1, True), (256, 256, 1, True)]
G = 4


def _pad_rows(w, rp):
    return jnp.pad(w, ((0, rp - w.shape[0]), (0, 0)))


def _prep(params):
    consts = []

    def add(a):
        consts.append(a.astype(jnp.float32))
        return len(consts) - 1

    def addh(a):
        consts.append(a.astype(jnp.bfloat16))
        return len(consts) - 1

    g = jnp.pad(params['data_bn']['g'].reshape(2, V, 3), ((0, 0), (0, VP - V), (0, 5)))
    b = jnp.pad(params['data_bn']['b'].reshape(2, V, 3), ((0, 0), (0, VP - V), (0, 5)))
    ig = add(jnp.tile(g, (G // 2, 1, 1)).reshape(G, 1, VP, 8))
    ib = add(jnp.tile(b, (G // 2, 1, 1)).reshape(G, 1, VP, 8))

    meta = []
    for (cin, cout, stride, residual), lp in zip(LCFG, params['layers']):
        gc, tc = lp['gcn'], lp['tcn']
        cp = 8 if cin == 3 else cin
        E = cout // 4
        m = {}
        wab = jnp.concatenate([_pad_rows(w, cp) for w in (gc['wa'] + gc['wb'])], axis=1)
        bab = jnp.concatenate(gc['ba'] + gc['bb']).reshape(1, 6 * E)
        m['wab'] = addh(wab)
        m['bab'] = add(bab)
        m['A'] = add(jnp.pad(gc['PA'], ((0, 0), (0, VP - V), (0, VP - V))))
        bng, bnb = gc['bn_g'], gc['bn_b']
        wd = jnp.concatenate([_pad_rows(w, cp) for w in gc['wd']], axis=0) * bng[None, :]
        bd = ((gc['bd'][0] + gc['bd'][1] + gc['bd'][2]) * bng + bnb).reshape(1, cout)
        m['wd'] = addh(wd)
        m['bd'] = add(bd)
        has_down = 'down_w' in gc
        if has_down:
            m['dw'] = addh(_pad_rows(gc['down_w'], cp) * gc['down_g'][None, :])
            m['db'] = add((gc['down_b'] * gc['down_g'] + gc['down_bb']).reshape(1, cout))
        tg = tc['g']
        wt = jnp.concatenate([tc['w'][:, :, k, 0].T * tg[None, :] for k in range(9)], axis=0)
        m['wt'] = addh(wt)
        m['bt'] = add((tc['b'] * tg + tc['bb']).reshape(1, cout))
        has_res = 'res' in lp
        if has_res:
            r = lp['res']
            m['rw'] = addh(r['w'][:, :, 0, 0].T * r['g'][None, :])
            m['rb'] = add((r['b'] * r['g'] + r['bb']).reshape(1, cout))
        meta.append((cp, cout, stride, residual, has_down, has_res, m))
    ifw = addh(params['fc_w'])
    ifb = add(params['fc_b'].reshape(1, NCLS))
    return consts, tuple(meta), (ig, ib, ifw, ifb)


def _body(meta, misc, x_ref, *refs):
    out_ref = refs[-1]
    cr = refs[:-1]
    ig, ib, ifw, ifb = misc
    B = G
    T = 64

    rowmask = (jax.lax.broadcasted_iota(jnp.int32, (VP, 1), 0) >= V)
    neg = jnp.where(rowmask, jnp.float32(-1e30), jnp.float32(0.0))
    colmask = (jax.lax.broadcasted_iota(jnp.int32, (1, VP), 1) < V).astype(jnp.float32)

    act = x_ref[...] * cr[ig][...] + cr[ib][...]

    for (cp, cout, stride, residual, has_down, has_res, m) in meta:
        E = cout // 4
        rows = B * T * VP
        X2 = act.reshape(rows, cp)
        X2h = X2.astype(jnp.bfloat16)
        a12 = jnp.dot(X2h, cr[m['wab']][...], preferred_element_type=jnp.float32)
        a12 = (a12 + cr[m['bab']][...]).astype(jnp.bfloat16)
        a12 = jnp.swapaxes(a12.reshape(B, T, VP, 6 * E), 2, 3).reshape(B, T, 6, E, VP)
        Apad = cr[m['A']][...]
        act_vm = jnp.swapaxes(X2h.reshape(B, T, VP, cp), 2, 3)
        Xv = act_vm.reshape(B, T * cp, VP)
        A1s = []
        for i in range(3):
            a1s = a12[:, :, i].reshape(B, T * E, VP)
            a2s = a12[:, :, 3 + i].reshape(B, T * E, VP)
            att = jax.lax.dot_general(a1s, a2s, (((1,), (1,)), ((0,), (0,))),
                                      preferred_element_type=jnp.float32)
            att = att * jnp.float32(1.0 / (E * T)) + neg
            mx = jnp.max(att, axis=1, keepdims=True)
            e = jnp.exp(att - mx)
            sm = e / jnp.sum(e, axis=1, keepdims=True)
            A1s.append(sm * colmask + Apad[i][None])
        A1cat = jnp.concatenate(A1s, axis=-1).astype(jnp.bfloat16)
        xab = jax.lax.dot_general(Xv, A1cat, (((2,), (1,)), ((0,), (0,))),
                                  preferred_element_type=jnp.float32)
        xab = xab.astype(jnp.bfloat16).reshape(B, T, cp, 3 * VP)
        xat = jnp.swapaxes(xab, 2, 3).reshape(B, T, 3, VP, cp)
        wd_all = cr[m['wd']][...]
        y = cr[m['bd']][...] * jnp.ones((rows, 1), jnp.float32)
        for i in range(3):
            xi = xat[:, :, i].reshape(rows, cp)
            y = y + jnp.dot(xi, wd_all[i * cp:(i + 1) * cp],
                            preferred_element_type=jnp.float32)
        if has_down:
            res = jnp.dot(X2h, cr[m['dw']][...], preferred_element_type=jnp.float32)
            res = res + cr[m['db']][...]
        else:
            res = X2
        gout = jax.nn.relu(y + res).astype(jnp.bfloat16).reshape(B, T, VP, cout)

        To = T // stride
        zer = jnp.zeros((B, 4, VP, cout), jnp.bfloat16)
        xp = jnp.concatenate([zer, gout, zer], axis=1)
        wt = cr[m['wt']][...]
        acc = cr[m['bt']][...].astype(jnp.float32) * jnp.ones((B * To * VP, 1), jnp.float32)
        if stride == 1:
            xcat = jnp.concatenate([xp[:, :-1], xp[:, 1:]], axis=-1)
            pair = [xcat[:, 2 * kp:2 * kp + T] for kp in range(4)]
            last = xp[:, 8:8 + T]
        else:
            xp2 = xp.reshape(B, (T + 8) // 2, 2, VP, cout)
            xcat = jnp.concatenate([xp2[:, :, 0], xp2[:, :, 1]], axis=-1)
            pair = [xcat[:, kp:kp + To] for kp in range(4)]
            last = xp2[:, 4:4 + To, 0]
        for kp in range(4):
            acc = acc + jnp.dot(pair[kp].reshape(B * To * VP, 2 * cout),
                                wt[2 * kp * cout:(2 * kp + 2) * cout],
                                preferred_element_type=jnp.float32)
        acc = acc + jnp.dot(last.reshape(B * To * VP, cout), wt[8 * cout:9 * cout],
                            preferred_element_type=jnp.float32)

        if not residual:
            nxt = jax.nn.relu(acc)
        elif has_res:
            if stride == 2:
                xs = act.reshape(B, T // 2, 2, VP, cp)[:, :, 0]
            else:
                xs = act
            res2 = jnp.dot(xs.reshape(B * To * VP, cp).astype(jnp.bfloat16),
                           cr[m['rw']][...],
                           preferred_element_type=jnp.float32) + cr[m['rb']][...]
            nxt = jax.nn.relu(acc + res2)
        else:
            nxt = jax.nn.relu(acc + X2)
        act = nxt.reshape(B, To, VP, cout)
        T = To

    vmask = (jax.lax.broadcasted_iota(jnp.int32, (VP, 1), 0) < V).astype(jnp.float32)
    s = jnp.sum(act * vmask, axis=1)
    s = jnp.sum(s, axis=1) * jnp.float32(1.0 / (T * V))
    pr = s.reshape(B // 2, 2, 256)
    pm = (pr[:, 0] + pr[:, 1]) * jnp.float32(0.5)
    out = jnp.dot(pm.astype(jnp.bfloat16), cr[ifw][...], preferred_element_type=jnp.float32)
    out_ref[0] = out + cr[ifb][...]


def kernel(x, params):
    consts, meta, misc = _prep(params)
    N, C0, T0, V0, M = x.shape
    xp = jnp.transpose(x, (0, 4, 2, 3, 1)).reshape(N * M, T0, V0, C0)
    xp = jnp.pad(xp, ((0, 0), (0, 0), (0, VP - V0), (0, 8 - C0)))
    B = N * M
    in_specs = [pl.BlockSpec((G, T0, VP, 8), lambda i: (i, 0, 0, 0))]
    for c in consts:
        nd = c.ndim
        in_specs.append(pl.BlockSpec(c.shape, (lambda nd_: lambda i: (0,) * nd_)(nd)))
    out = pl.pallas_call(
        functools.partial(_body, meta, misc),
        grid=(B // G,),
        in_specs=in_specs,
        out_specs=pl.BlockSpec((1, G // 2, NCLS), lambda i: (i, 0, 0)),
        out_shape=jax.ShapeDtypeStruct((B // G, G // 2, NCLS), jnp.float32),
        compiler_params=pltpu.CompilerParams(dimension_semantics=("arbitrary",)),
    )(xp, *consts)
    return out.reshape(N, NCLS)
